# baseline (speedup 1.0000x reference)
import functools

import jax
import jax.numpy as jnp
from jax import lax
from jax.experimental import pallas as pl
from jax.experimental.pallas import tpu as pltpu
from jax.experimental.pallas import tpu_sc as plsc

VOCAB = 100000
HIDDEN = 1024
MAX_POS = 4096
BATCH = 4
SEQ = 4096
EPS = 1e-12

NW = 32
C = 16
SPW = SEQ // NW
NJ = SPW // C
L = 16
HV = HIDDEN // L
NCH = NJ * BATCH


def _rsqrt(v):
    half = v * 0.5
    i = plsc.bitcast(v, jnp.int32)
    i = jnp.int32(0x5F3759DF) - (i >> 1)
    y = plsc.bitcast(i, jnp.float32)
    y = y * (1.5 - half * y * y)
    y = y * (1.5 - half * y * y)
    y = y * (1.5 - half * y * y)
    return y


def _sc_embed_kernel(idx_hbm, words_hbm, pos_hbm, lnw_hbm, out_hbm,
                     idx_v, pbuf, wbuf, lnw_v, gsem, ssem, psem):
    wid = lax.axis_index("s") * 2 + lax.axis_index("c")
    p0 = wid * SPW

    pltpu.sync_copy(lnw_hbm, lnw_v)
    pltpu.sync_copy(idx_hbm.at[wid], idx_v)

    def pos_src(j):
        return pos_hbm.at[pl.ds(p0 + j * C, C)]

    def out_dst(i):
        off = (i & 3) * SEQ + p0 + (i >> 2) * C
        return out_hbm.at[pl.ds(off, C)]

    pltpu.async_copy(pos_src(0), pbuf.at[0], psem.at[0])
    pltpu.async_copy(words_hbm.at[idx_v.at[0]], wbuf.at[0], gsem.at[0])

    def body(i, carry):
        j = i >> 2
        b = i & 3

        @pl.when(b == 0)
        def _():
            pltpu.make_async_copy(pos_src(j), pbuf.at[j & 1],
                                  psem.at[j & 1]).wait()

            @pl.when(j + 1 < NJ)
            def _():
                pltpu.async_copy(pos_src(j + 1), pbuf.at[(j + 1) & 1],
                                 psem.at[(j + 1) & 1])

        @pl.when(i + 1 < NCH)
        def _():
            @pl.when(i + 1 >= 4)
            def _():
                pltpu.make_async_copy(wbuf.at[(i + 1) & 3], out_dst(i - 3),
                                      ssem.at[(i + 1) & 3]).wait()

            pltpu.async_copy(words_hbm.at[idx_v.at[i + 1]],
                             wbuf.at[(i + 1) & 3], gsem.at[(i + 1) & 3])

        pltpu.make_async_copy(words_hbm.at[idx_v.at[i]], wbuf.at[b],
                              gsem.at[b]).wait()

        def tbody(t, tc):
            acc = jnp.zeros((L,), jnp.float32)
            for h in range(HV):
                w = wbuf[b, t, pl.ds(h * L, L)]
                p = pbuf[j & 1, t, pl.ds(h * L, L)]
                x = w + p
                wbuf[b, t, pl.ds(h * L, L)] = x
                acc = acc + x * x
            r = jnp.sum(acc) * (1.0 / HIDDEN) + EPS
            s = _rsqrt(jnp.broadcast_to(r, (L,)))
            for h in range(HV):
                x = wbuf[b, t, pl.ds(h * L, L)]
                wbuf[b, t, pl.ds(h * L, L)] = x * s * lnw_v[pl.ds(h * L, L)]
            return tc

        lax.fori_loop(0, C, tbody, 0)

        pltpu.async_copy(wbuf.at[b], out_dst(i), ssem.at[b])
        return carry

    lax.fori_loop(0, NCH, body, 0)

    for u in range(4):
        i = NCH - 4 + u
        pltpu.make_async_copy(wbuf.at[i & 3], out_dst(i),
                              ssem.at[i & 3]).wait()


@jax.jit
def _sc_embed(idx, words, pos, lnw):
    mesh = plsc.VectorSubcoreMesh(core_axis_name="c", subcore_axis_name="s")
    f = functools.partial(
        pl.kernel,
        mesh=mesh,
        out_type=jax.ShapeDtypeStruct((BATCH * SEQ, HIDDEN), jnp.float32),
        scratch_types=[
            pltpu.VMEM((NCH, C), jnp.int32),
            pltpu.VMEM((2, C, HIDDEN), jnp.float32),
            pltpu.VMEM((4, C, HIDDEN), jnp.float32),
            pltpu.VMEM((HIDDEN,), jnp.float32),
            pltpu.SemaphoreType.DMA((4,)),
            pltpu.SemaphoreType.DMA((4,)),
            pltpu.SemaphoreType.DMA((2,)),
        ],
        compiler_params=pltpu.CompilerParams(needs_layout_passes=False),
    )(_sc_embed_kernel)
    return f(idx, words, pos, lnw)


def kernel(input_ids, word_embeddings, position_embeddings, ln_weight):
    ids = input_ids.astype(jnp.int32)
    idx = ids.reshape(BATCH, NW, NJ, C).transpose(1, 2, 0, 3)
    idx = idx.reshape(NW, NJ * BATCH, C)
    out = _sc_embed(idx, word_embeddings, position_embeddings, ln_weight)
    return out.reshape(BATCH, SEQ, HIDDEN)


# speedup vs baseline: 1.1421x; 1.1421x over previous
import functools

import jax
import jax.numpy as jnp
from jax import lax
from jax.experimental import pallas as pl
from jax.experimental.pallas import tpu as pltpu
from jax.experimental.pallas import tpu_sc as plsc

VOCAB = 100000
HIDDEN = 1024
MAX_POS = 4096
BATCH = 4
SEQ = 4096
EPS = 1e-12

NW = 32
C = 16
SPW = SEQ // NW
NJ = SPW // C
L = 16
HV = HIDDEN // L
NCH = NJ * BATCH


def _rsqrt(v):
    half = v * 0.5
    i = plsc.bitcast(v, jnp.int32)
    i = jnp.int32(0x5F3759DF) - (i >> 1)
    y = plsc.bitcast(i, jnp.float32)
    y = y * (1.5 - half * y * y)
    y = y * (1.5 - half * y * y)
    y = y * (1.5 - half * y * y)
    return y


def _sc_embed_kernel(idx_hbm, words_hbm, pos_hbm, lnw_hbm, out_hbm,
                     idx_v, pbuf, wbuf, xbuf, obuf, lnw_v, gsem, ssem, psem):
    wid = lax.axis_index("s") * 2 + lax.axis_index("c")
    p0 = wid * SPW

    pltpu.sync_copy(lnw_hbm, lnw_v)
    pltpu.sync_copy(idx_hbm.at[wid], idx_v)

    def pos_src(j):
        return pos_hbm.at[pl.ds(p0 + j * C, C)]

    def out_dst(i):
        off = (i & 3) * SEQ + p0 + (i >> 2) * C
        return out_hbm.at[pl.ds(off, C)]

    pltpu.async_copy(pos_src(0), pbuf.at[0], psem.at[0])
    pltpu.async_copy(words_hbm.at[idx_v.at[0]], wbuf.at[0], gsem.at[0])

    def body(i, carry):
        cur = i & 1
        j = i >> 2
        jc = j & 1

        @pl.when((i & 3) == 0)
        def _():
            pltpu.make_async_copy(pos_src(j), pbuf.at[jc], psem.at[jc]).wait()

            @pl.when(j + 1 < NJ)
            def _():
                pltpu.async_copy(pos_src(j + 1), pbuf.at[1 - jc],
                                 psem.at[1 - jc])

        @pl.when(i + 1 < NCH)
        def _():
            pltpu.async_copy(words_hbm.at[idx_v.at[i + 1]],
                             wbuf.at[1 - cur], gsem.at[1 - cur])

        pltpu.make_async_copy(words_hbm.at[idx_v.at[i]], wbuf.at[cur],
                              gsem.at[cur]).wait()

        @pl.when(i >= 2)
        def _():
            pltpu.make_async_copy(obuf.at[cur], out_dst(i - 2),
                                  ssem.at[cur]).wait()

        def tbody(t, tc):
            a0 = jnp.zeros((L,), jnp.float32)
            a1 = jnp.zeros((L,), jnp.float32)
            a2 = jnp.zeros((L,), jnp.float32)
            a3 = jnp.zeros((L,), jnp.float32)
            for h in range(0, HV, 4):
                x0 = wbuf[cur, t, pl.ds(h * L, L)] + pbuf[jc, t, pl.ds(h * L, L)]
                x1 = wbuf[cur, t, pl.ds((h + 1) * L, L)] + pbuf[jc, t, pl.ds((h + 1) * L, L)]
                x2 = wbuf[cur, t, pl.ds((h + 2) * L, L)] + pbuf[jc, t, pl.ds((h + 2) * L, L)]
                x3 = wbuf[cur, t, pl.ds((h + 3) * L, L)] + pbuf[jc, t, pl.ds((h + 3) * L, L)]
                xbuf[t, pl.ds(h * L, L)] = x0
                xbuf[t, pl.ds((h + 1) * L, L)] = x1
                xbuf[t, pl.ds((h + 2) * L, L)] = x2
                xbuf[t, pl.ds((h + 3) * L, L)] = x3
                a0 = a0 + x0 * x0
                a1 = a1 + x1 * x1
                a2 = a2 + x2 * x2
                a3 = a3 + x3 * x3
            acc = (a0 + a1) + (a2 + a3)
            r = jnp.sum(acc) * (1.0 / HIDDEN) + EPS
            s = _rsqrt(jnp.broadcast_to(r, (L,)))
            for h in range(HV):
                x = xbuf[t, pl.ds(h * L, L)]
                obuf[cur, t, pl.ds(h * L, L)] = x * s * lnw_v[pl.ds(h * L, L)]
            return tc

        lax.fori_loop(0, C, tbody, 0)

        pltpu.async_copy(obuf.at[cur], out_dst(i), ssem.at[cur])
        return carry

    lax.fori_loop(0, NCH, body, 0)

    for u in range(2):
        i = NCH - 2 + u
        pltpu.make_async_copy(obuf.at[i & 1], out_dst(i), ssem.at[i & 1]).wait()


@jax.jit
def _sc_embed(idx, words, pos, lnw):
    mesh = plsc.VectorSubcoreMesh(core_axis_name="c", subcore_axis_name="s")
    f = functools.partial(
        pl.kernel,
        mesh=mesh,
        out_type=jax.ShapeDtypeStruct((BATCH * SEQ, HIDDEN), jnp.float32),
        scratch_types=[
            pltpu.VMEM((NCH, C), jnp.int32),
            pltpu.VMEM((2, C, HIDDEN), jnp.float32),
            pltpu.VMEM((2, C, HIDDEN), jnp.float32),
            pltpu.VMEM((C, HIDDEN), jnp.float32),
            pltpu.VMEM((2, C, HIDDEN), jnp.float32),
            pltpu.VMEM((HIDDEN,), jnp.float32),
            pltpu.SemaphoreType.DMA((2,)),
            pltpu.SemaphoreType.DMA((2,)),
            pltpu.SemaphoreType.DMA((2,)),
        ],
        compiler_params=pltpu.CompilerParams(needs_layout_passes=False),
    )(_sc_embed_kernel)
    return f(idx, words, pos, lnw)


def kernel(input_ids, word_embeddings, position_embeddings, ln_weight):
    ids = input_ids.astype(jnp.int32)
    idx = ids.reshape(BATCH, NW, NJ, C).transpose(1, 2, 0, 3)
    idx = idx.reshape(NW, NJ * BATCH, C)
    out = _sc_embed(idx, word_embeddings, position_embeddings, ln_weight)
    return out.reshape(BATCH, SEQ, HIDDEN)


# speedup vs baseline: 2.8657x; 2.5091x over previous
import functools

import jax
import jax.numpy as jnp
from jax import lax
from jax.experimental import pallas as pl
from jax.experimental.pallas import tpu as pltpu
from jax.experimental.pallas import tpu_sc as plsc

VOCAB = 100000
HIDDEN = 1024
MAX_POS = 4096
BATCH = 4
SEQ = 4096
EPS = 1e-12

NW = 32
C = 8
SPW = SEQ // NW
NJ = SPW // C
L = 16
HV = HIDDEN // L
HB = 4


def _rsqrt(v):
    half = v * 0.5
    i = plsc.bitcast(v, jnp.int32)
    i = jnp.int32(0x5F3759DF) - (i >> 1)
    y = plsc.bitcast(i, jnp.float32)
    y = y * (1.5 - half * y * y)
    y = y * (1.5 - half * y * y)
    y = y * (1.5 - half * y * y)
    return y


def _lane_bcast(v, lane):
    idx = jnp.full((L,), lane, dtype=jnp.int32)
    return lax.gather(
        v, idx[:, None],
        dimension_numbers=lax.GatherDimensionNumbers(
            offset_dims=(), collapsed_slice_dims=(0,), start_index_map=(0,)),
        slice_sizes=(1,),
        mode=lax.GatherScatterMode.PROMISE_IN_BOUNDS)


def _sc_embed_kernel(idx_hbm, words_hbm, pos_hbm, lnw_hbm, out_hbm,
                     idx_v, pbuf, wbuf, obuf, lnw_v, gsem, ssem, psem):
    wid = lax.axis_index("s") * 2 + lax.axis_index("c")
    p0 = wid * SPW

    pltpu.sync_copy(lnw_hbm, lnw_v)
    pltpu.sync_copy(idx_hbm.at[wid], idx_v)

    lane = lax.iota(jnp.int32, L)

    def pos_src(j):
        return pos_hbm.at[pl.ds(p0 + j * C, C)]

    def out_dst(j, b):
        return out_hbm.at[pl.ds(b * SEQ + p0 + j * C, C)]

    def start_gathers(j, ring):
        for b in range(BATCH):
            pltpu.async_copy(words_hbm.at[idx_v.at[j * BATCH + b]],
                             wbuf.at[ring, b], gsem.at[ring, b])

    pltpu.async_copy(pos_src(0), pbuf.at[0], psem.at[0])
    start_gathers(0, 0)
    pltpu.async_copy(pos_src(1), pbuf.at[1], psem.at[1])

    def body(j, carry):
        ring = j & 1
        pr = j & 1

        @pl.when(j + 1 < NJ)
        def _():
            start_gathers(j + 1, 1 - ring)

        pltpu.make_async_copy(pos_src(j), pbuf.at[pr], psem.at[pr]).wait()

        @pl.when(j + 2 < NJ)
        def _():
            pltpu.async_copy(pos_src(j + 2), pbuf.at[1 - pr],
                             psem.at[1 - pr])

        for b in range(BATCH):
            pltpu.make_async_copy(words_hbm.at[idx_v.at[j * BATCH + b]],
                                  wbuf.at[ring, b], gsem.at[ring, b]).wait()

        @pl.when(j >= 1)
        def _():
            for b in range(BATCH):
                pltpu.make_async_copy(obuf.at[b], out_dst(j - 1, b),
                                      ssem.at[b]).wait()

        def tbody(t, tc):
            acc = [jnp.zeros((L,), jnp.float32) for _ in range(2 * BATCH)]
            for h in range(HV):
                sl = pl.ds(h * L, L)
                p = pbuf[pr, t, sl]
                for b in range(BATCH):
                    x = wbuf[ring, b, t, sl] + p
                    acc[2 * b + (h & 1)] = acc[2 * b + (h & 1)] + x * x
            r = None
            for b in range(BATCH):
                tot = plsc.cumsum(acc[2 * b] + acc[2 * b + 1])
                tot = _lane_bcast(tot, L - 1)
                r = tot if b == 0 else jnp.where(lane == b, tot, r)
            s = _rsqrt(r * (1.0 / HIDDEN) + EPS)
            sb = [_lane_bcast(s, b) for b in range(BATCH)]
            for h0 in range(0, HV, HB):
                ps, ls, ws = [], [], []
                for k in range(HB):
                    sl = pl.ds((h0 + k) * L, L)
                    ps.append(pbuf[pr, t, sl])
                    ls.append(lnw_v[sl])
                    for b in range(BATCH):
                        ws.append(wbuf[ring, b, t, sl])
                ys = []
                for k in range(HB):
                    for b in range(BATCH):
                        x = ws[k * BATCH + b] + ps[k]
                        ys.append(x * sb[b] * ls[k])
                for k in range(HB):
                    sl = pl.ds((h0 + k) * L, L)
                    for b in range(BATCH):
                        obuf[b, t, sl] = ys[k * BATCH + b]
            return tc

        lax.fori_loop(0, C, tbody, 0)

        for b in range(BATCH):
            pltpu.async_copy(obuf.at[b], out_dst(j, b), ssem.at[b])
        return carry

    lax.fori_loop(0, NJ, body, 0)

    for b in range(BATCH):
        pltpu.make_async_copy(obuf.at[b], out_dst(NJ - 1, b),
                              ssem.at[b]).wait()


@jax.jit
def _sc_embed(idx, words, pos, lnw):
    mesh = plsc.VectorSubcoreMesh(core_axis_name="c", subcore_axis_name="s")
    f = functools.partial(
        pl.kernel,
        mesh=mesh,
        out_type=jax.ShapeDtypeStruct((BATCH * SEQ, HIDDEN), jnp.float32),
        scratch_types=[
            pltpu.VMEM((NJ * BATCH, C), jnp.int32),
            pltpu.VMEM((2, C, HIDDEN), jnp.float32),
            pltpu.VMEM((2, BATCH, C, HIDDEN), jnp.float32),
            pltpu.VMEM((BATCH, C, HIDDEN), jnp.float32),
            pltpu.VMEM((HIDDEN,), jnp.float32),
            pltpu.SemaphoreType.DMA((2, BATCH)),
            pltpu.SemaphoreType.DMA((BATCH,)),
            pltpu.SemaphoreType.DMA((2,)),
        ],
        compiler_params=pltpu.CompilerParams(needs_layout_passes=False),
    )(_sc_embed_kernel)
    return f(idx, words, pos, lnw)


def kernel(input_ids, word_embeddings, position_embeddings, ln_weight):
    ids = input_ids.astype(jnp.int32)
    idx = ids.reshape(BATCH, NW, NJ, C).transpose(1, 2, 0, 3)
    idx = idx.reshape(NW, NJ * BATCH, C)
    out = _sc_embed(idx, word_embeddings, position_embeddings, ln_weight)
    return out.reshape(BATCH, SEQ, HIDDEN)


# speedup vs baseline: 2.8738x; 1.0028x over previous
import functools

import jax
import jax.numpy as jnp
from jax import lax
from jax.experimental import pallas as pl
from jax.experimental.pallas import tpu as pltpu
from jax.experimental.pallas import tpu_sc as plsc

VOCAB = 100000
HIDDEN = 1024
MAX_POS = 4096
BATCH = 4
SEQ = 4096
EPS = 1e-12

NW = 32
C = 8
SPW = SEQ // NW
NJ = SPW // C
L = 16
HV = HIDDEN // L
HB = 4


def _rsqrt(v):
    half = v * 0.5
    i = plsc.bitcast(v, jnp.int32)
    i = jnp.int32(0x5F3759DF) - (i >> 1)
    y = plsc.bitcast(i, jnp.float32)
    y = y * (1.5 - half * y * y)
    y = y * (1.5 - half * y * y)
    y = y * (1.5 - half * y * y)
    return y


def _lane_bcast(v, lane):
    idx = jnp.full((L,), lane, dtype=jnp.int32)
    return lax.gather(
        v, idx[:, None],
        dimension_numbers=lax.GatherDimensionNumbers(
            offset_dims=(), collapsed_slice_dims=(0,), start_index_map=(0,)),
        slice_sizes=(1,),
        mode=lax.GatherScatterMode.PROMISE_IN_BOUNDS)


def _sc_embed_kernel(idx_hbm, words_hbm, pos_hbm, lnw_hbm, out_hbm,
                     idx_v, pbuf, wbuf, obuf, lnw_v, gsem, ssem, psem):
    wid = lax.axis_index("s") * 2 + lax.axis_index("c")
    p0 = wid * SPW

    pltpu.sync_copy(lnw_hbm, lnw_v)
    pltpu.sync_copy(idx_hbm.at[wid], idx_v)

    lane = lax.iota(jnp.int32, L)

    def pos_src(j):
        return pos_hbm.at[pl.ds(p0 + j * C, C)]

    def out_dst(j, b):
        return out_hbm.at[pl.ds(b * SEQ + p0 + j * C, C)]

    def start_gathers(j, ring):
        for b in range(BATCH):
            pltpu.async_copy(words_hbm.at[idx_v.at[j * BATCH + b]],
                             wbuf.at[ring, b], gsem.at[ring, b])

    pltpu.async_copy(pos_src(0), pbuf.at[0], psem.at[0])
    start_gathers(0, 0)

    def body(j, carry):
        ring = j & 1
        pr = j & 1

        @pl.when(j + 1 < NJ)
        def _():
            start_gathers(j + 1, 1 - ring)

        pltpu.make_async_copy(pos_src(j), pbuf.at[pr], psem.at[pr]).wait()

        @pl.when(j + 1 < NJ)
        def _():
            pltpu.async_copy(pos_src(j + 1), pbuf.at[1 - pr],
                             psem.at[1 - pr])

        for b in range(BATCH):
            pltpu.make_async_copy(words_hbm.at[idx_v.at[j * BATCH + b]],
                                  wbuf.at[ring, b], gsem.at[ring, b]).wait()

        @pl.when(j >= 1)
        def _():
            for b in range(BATCH):
                pltpu.make_async_copy(obuf.at[b], out_dst(j - 1, b),
                                      ssem.at[b]).wait()

        def tbody(t, tc):
            acc = [jnp.zeros((L,), jnp.float32) for _ in range(2 * BATCH)]
            for h in range(HV):
                sl = pl.ds(h * L, L)
                p = pbuf[pr, t, sl]
                for b in range(BATCH):
                    x = wbuf[ring, b, t, sl] + p
                    acc[2 * b + (h & 1)] = acc[2 * b + (h & 1)] + x * x
            r = None
            for b in range(BATCH):
                tot = plsc.cumsum(acc[2 * b] + acc[2 * b + 1])
                tot = _lane_bcast(tot, L - 1)
                r = tot if b == 0 else jnp.where(lane == b, tot, r)
            s = _rsqrt(r * (1.0 / HIDDEN) + EPS)
            sb = [_lane_bcast(s, b) for b in range(BATCH)]
            for h0 in range(0, HV, HB):
                ps, ls, ws = [], [], []
                for k in range(HB):
                    sl = pl.ds((h0 + k) * L, L)
                    ps.append(pbuf[pr, t, sl])
                    ls.append(lnw_v[sl])
                    for b in range(BATCH):
                        ws.append(wbuf[ring, b, t, sl])
                ys = []
                for k in range(HB):
                    for b in range(BATCH):
                        x = ws[k * BATCH + b] + ps[k]
                        ys.append(x * sb[b] * ls[k])
                for k in range(HB):
                    sl = pl.ds((h0 + k) * L, L)
                    for b in range(BATCH):
                        obuf[b, t, sl] = ys[k * BATCH + b]
            return tc

        lax.fori_loop(0, C, tbody, 0)

        for b in range(BATCH):
            pltpu.async_copy(obuf.at[b], out_dst(j, b), ssem.at[b])
        return carry

    lax.fori_loop(0, NJ, body, 0)

    for b in range(BATCH):
        pltpu.make_async_copy(obuf.at[b], out_dst(NJ - 1, b),
                              ssem.at[b]).wait()


@jax.jit
def _sc_embed(idx, words, pos, lnw):
    mesh = plsc.VectorSubcoreMesh(core_axis_name="c", subcore_axis_name="s")
    f = functools.partial(
        pl.kernel,
        mesh=mesh,
        out_type=jax.ShapeDtypeStruct((BATCH * SEQ, HIDDEN), jnp.float32),
        scratch_types=[
            pltpu.VMEM((NJ * BATCH, C), jnp.int32),
            pltpu.VMEM((2, C, HIDDEN), jnp.float32),
            pltpu.VMEM((2, BATCH, C, HIDDEN), jnp.float32),
            pltpu.VMEM((BATCH, C, HIDDEN), jnp.float32),
            pltpu.VMEM((HIDDEN,), jnp.float32),
            pltpu.SemaphoreType.DMA((2, BATCH)),
            pltpu.SemaphoreType.DMA((BATCH,)),
            pltpu.SemaphoreType.DMA((2,)),
        ],
        compiler_params=pltpu.CompilerParams(needs_layout_passes=False),
    )(_sc_embed_kernel)
    return f(idx, words, pos, lnw)


def kernel(input_ids, word_embeddings, position_embeddings, ln_weight):
    ids = input_ids.astype(jnp.int32)
    idx = ids.reshape(BATCH, NW, NJ, C).transpose(1, 2, 0, 3)
    idx = idx.reshape(NW, NJ * BATCH, C)
    out = _sc_embed(idx, word_embeddings, position_embeddings, ln_weight)
    return out.reshape(BATCH, SEQ, HIDDEN)


# speedup vs baseline: 2.9234x; 1.0173x over previous
import functools

import jax
import jax.numpy as jnp
from jax import lax
from jax.experimental import pallas as pl
from jax.experimental.pallas import tpu as pltpu
from jax.experimental.pallas import tpu_sc as plsc

VOCAB = 100000
HIDDEN = 1024
MAX_POS = 4096
BATCH = 4
SEQ = 4096
EPS = 1e-12

NW = 32
C = 8
SPW = SEQ // NW
NJ = SPW // C
L = 16
HV = HIDDEN // L
HB = 4


def _rsqrt(v):
    half = v * 0.5
    i = plsc.bitcast(v, jnp.int32)
    i = jnp.int32(0x5F3759DF) - (i >> 1)
    y = plsc.bitcast(i, jnp.float32)
    y = y * (1.5 - half * y * y)
    y = y * (1.5 - half * y * y)
    y = y * (1.5 - half * y * y)
    return y


def _lane_bcast(v, lane):
    idx = jnp.full((L,), lane, dtype=jnp.int32)
    return lax.gather(
        v, idx[:, None],
        dimension_numbers=lax.GatherDimensionNumbers(
            offset_dims=(), collapsed_slice_dims=(0,), start_index_map=(0,)),
        slice_sizes=(1,),
        mode=lax.GatherScatterMode.PROMISE_IN_BOUNDS)


def _sc_embed_kernel(idx_hbm, words_hbm, pos_hbm, lnw_hbm, out_hbm,
                     idx_v, pbuf, wbuf, obuf, lnw_v, gsem, ssem, psem):
    wid = lax.axis_index("s") * 2 + lax.axis_index("c")
    p0 = wid * SPW

    pltpu.sync_copy(lnw_hbm, lnw_v)
    pltpu.sync_copy(idx_hbm.at[wid], idx_v)

    lane = lax.iota(jnp.int32, L)

    ones_chk = lnw_v[pl.ds(0, L)] == 1.0
    for h in range(1, HV):
        ones_chk = jnp.logical_and(ones_chk, lnw_v[pl.ds(h * L, L)] == 1.0)
    lnw_ones = jnp.all(ones_chk)

    def pos_src(j):
        return pos_hbm.at[pl.ds(p0 + j * C, C)]

    def out_dst(j, b):
        return out_hbm.at[pl.ds(b * SEQ + p0 + j * C, C)]

    def start_gathers(j, ring):
        for b in range(BATCH):
            pltpu.async_copy(words_hbm.at[idx_v.at[j * BATCH + b]],
                             wbuf.at[ring, b], gsem.at[ring, b])

    pltpu.async_copy(pos_src(0), pbuf.at[0], psem.at[0])
    start_gathers(0, 0)

    def body(j, carry):
        ring = j & 1
        pr = j & 1

        @pl.when(j + 1 < NJ)
        def _():
            start_gathers(j + 1, 1 - ring)

        pltpu.make_async_copy(pos_src(j), pbuf.at[pr], psem.at[pr]).wait()

        @pl.when(j + 1 < NJ)
        def _():
            pltpu.async_copy(pos_src(j + 1), pbuf.at[1 - pr],
                             psem.at[1 - pr])

        for b in range(BATCH):
            pltpu.make_async_copy(words_hbm.at[idx_v.at[j * BATCH + b]],
                                  wbuf.at[ring, b], gsem.at[ring, b]).wait()

        @pl.when(j >= 1)
        def _():
            for b in range(BATCH):
                pltpu.make_async_copy(obuf.at[b], out_dst(j - 1, b),
                                      ssem.at[b]).wait()

        def make_tbody(with_lnw):
            def tbody(t, tc):
                acc = [jnp.zeros((L,), jnp.float32) for _ in range(2 * BATCH)]
                for h in range(HV):
                    sl = pl.ds(h * L, L)
                    p = pbuf[pr, t, sl]
                    for b in range(BATCH):
                        x = wbuf[ring, b, t, sl] + p
                        acc[2 * b + (h & 1)] = acc[2 * b + (h & 1)] + x * x
                r = None
                for b in range(BATCH):
                    tot = plsc.cumsum(acc[2 * b] + acc[2 * b + 1])
                    tot = _lane_bcast(tot, L - 1)
                    r = tot if b == 0 else jnp.where(lane == b, tot, r)
                s = _rsqrt(r * (1.0 / HIDDEN) + EPS)
                sb = [_lane_bcast(s, b) for b in range(BATCH)]
                for h0 in range(0, HV, HB):
                    ps, ls, ws = [], [], []
                    for k in range(HB):
                        sl = pl.ds((h0 + k) * L, L)
                        ps.append(pbuf[pr, t, sl])
                        if with_lnw:
                            ls.append(lnw_v[sl])
                        for b in range(BATCH):
                            ws.append(wbuf[ring, b, t, sl])
                    ys = []
                    for k in range(HB):
                        for b in range(BATCH):
                            x = ws[k * BATCH + b] + ps[k]
                            y = x * sb[b]
                            ys.append(y * ls[k] if with_lnw else y)
                    for k in range(HB):
                        sl = pl.ds((h0 + k) * L, L)
                        for b in range(BATCH):
                            obuf[b, t, sl] = ys[k * BATCH + b]
                return tc
            return tbody

        @pl.when(lnw_ones)
        def _():
            lax.fori_loop(0, C, make_tbody(False), 0)

        @pl.when(jnp.logical_not(lnw_ones))
        def _():
            lax.fori_loop(0, C, make_tbody(True), 0)

        for b in range(BATCH):
            pltpu.async_copy(obuf.at[b], out_dst(j, b), ssem.at[b])
        return carry

    lax.fori_loop(0, NJ, body, 0)

    for b in range(BATCH):
        pltpu.make_async_copy(obuf.at[b], out_dst(NJ - 1, b),
                              ssem.at[b]).wait()


@jax.jit
def _sc_embed(idx, words, pos, lnw):
    mesh = plsc.VectorSubcoreMesh(core_axis_name="c", subcore_axis_name="s")
    f = functools.partial(
        pl.kernel,
        mesh=mesh,
        out_type=jax.ShapeDtypeStruct((BATCH * SEQ, HIDDEN), jnp.float32),
        scratch_types=[
            pltpu.VMEM((NJ * BATCH, C), jnp.int32),
            pltpu.VMEM((2, C, HIDDEN), jnp.float32),
            pltpu.VMEM((2, BATCH, C, HIDDEN), jnp.float32),
            pltpu.VMEM((BATCH, C, HIDDEN), jnp.float32),
            pltpu.VMEM((HIDDEN,), jnp.float32),
            pltpu.SemaphoreType.DMA((2, BATCH)),
            pltpu.SemaphoreType.DMA((BATCH,)),
            pltpu.SemaphoreType.DMA((2,)),
        ],
        compiler_params=pltpu.CompilerParams(needs_layout_passes=False),
    )(_sc_embed_kernel)
    return f(idx, words, pos, lnw)


def kernel(input_ids, word_embeddings, position_embeddings, ln_weight):
    ids = input_ids.astype(jnp.int32)
    idx = ids.reshape(BATCH, NW, NJ, C).transpose(1, 2, 0, 3)
    idx = idx.reshape(NW, NJ * BATCH, C)
    out = _sc_embed(idx, word_embeddings, position_embeddings, ln_weight)
    return out.reshape(BATCH, SEQ, HIDDEN)


# speedup vs baseline: 2.9352x; 1.0040x over previous
import functools

import jax
import jax.numpy as jnp
from jax import lax
from jax.experimental import pallas as pl
from jax.experimental.pallas import tpu as pltpu
from jax.experimental.pallas import tpu_sc as plsc

VOCAB = 100000
HIDDEN = 1024
MAX_POS = 4096
BATCH = 4
SEQ = 4096
EPS = 1e-12

NW = 32
C = 8
SPW = SEQ // NW
NJ = SPW // C
L = 16
HV = HIDDEN // L
HB = 4


def _rsqrt(v):
    half = v * 0.5
    i = plsc.bitcast(v, jnp.int32)
    i = jnp.int32(0x5F3759DF) - (i >> 1)
    y = plsc.bitcast(i, jnp.float32)
    y = y * (1.5 - half * y * y)
    y = y * (1.5 - half * y * y)
    return y


def _perm(v, idx):
    return lax.gather(
        v, idx[:, None],
        dimension_numbers=lax.GatherDimensionNumbers(
            offset_dims=(), collapsed_slice_dims=(0,), start_index_map=(0,)),
        slice_sizes=(1,),
        mode=lax.GatherScatterMode.PROMISE_IN_BOUNDS)


def _lane_bcast(v, lane):
    return _perm(v, jnp.full((L,), lane, dtype=jnp.int32))


def _allsum(v, lane):
    for d in (8, 4, 2, 1):
        v = v + _perm(v, lane ^ d)
    return v


def _sc_embed_kernel(idx_hbm, words_hbm, pos_hbm, lnw_hbm, out_hbm,
                     idx_v, pbuf, wbuf, obuf, lnw_v, gsem, ssem, psem):
    wid = lax.axis_index("s") * 2 + lax.axis_index("c")
    p0 = wid * SPW

    pltpu.sync_copy(lnw_hbm, lnw_v)
    pltpu.sync_copy(idx_hbm.at[wid], idx_v)

    lane = lax.iota(jnp.int32, L)

    ones_chk = lnw_v[pl.ds(0, L)] == 1.0
    for h in range(1, HV):
        ones_chk = jnp.logical_and(ones_chk, lnw_v[pl.ds(h * L, L)] == 1.0)
    lnw_ones = jnp.all(ones_chk)

    def pos_src(j):
        return pos_hbm.at[pl.ds(p0 + j * C, C)]

    def out_dst(j, b):
        return out_hbm.at[pl.ds(b * SEQ + p0 + j * C, C)]

    def start_gathers(j, ring):
        for b in range(BATCH):
            pltpu.async_copy(words_hbm.at[idx_v.at[j * BATCH + b]],
                             wbuf.at[ring, b], gsem.at[ring, b])

    pltpu.async_copy(pos_src(0), pbuf.at[0], psem.at[0])
    start_gathers(0, 0)

    def body(j, carry):
        ring = j & 1
        pr = j & 1

        @pl.when(j + 1 < NJ)
        def _():
            start_gathers(j + 1, 1 - ring)

        pltpu.make_async_copy(pos_src(j), pbuf.at[pr], psem.at[pr]).wait()

        @pl.when(j + 1 < NJ)
        def _():
            pltpu.async_copy(pos_src(j + 1), pbuf.at[1 - pr],
                             psem.at[1 - pr])

        for b in range(BATCH):
            pltpu.make_async_copy(words_hbm.at[idx_v.at[j * BATCH + b]],
                                  wbuf.at[ring, b], gsem.at[ring, b]).wait()

        @pl.when(j >= 1)
        def _():
            for b in range(BATCH):
                pltpu.make_async_copy(obuf.at[b], out_dst(j - 1, b),
                                      ssem.at[b]).wait()

        def make_tbody(with_lnw):
            def tbody(t, tc):
                acc = [jnp.zeros((L,), jnp.float32) for _ in range(2 * BATCH)]
                for h in range(HV):
                    sl = pl.ds(h * L, L)
                    p = pbuf[pr, t, sl]
                    for b in range(BATCH):
                        x = wbuf[ring, b, t, sl] + p
                        acc[2 * b + (h & 1)] = acc[2 * b + (h & 1)] + x * x
                r = None
                for b in range(BATCH):
                    tot = _allsum(acc[2 * b] + acc[2 * b + 1], lane)
                    r = tot if b == 0 else jnp.where(lane == b, tot, r)
                s = _rsqrt(r * (1.0 / HIDDEN) + EPS)
                sb = [_lane_bcast(s, b) for b in range(BATCH)]
                for h0 in range(0, HV, HB):
                    ps, ls, ws = [], [], []
                    for k in range(HB):
                        sl = pl.ds((h0 + k) * L, L)
                        ps.append(pbuf[pr, t, sl])
                        if with_lnw:
                            ls.append(lnw_v[sl])
                        for b in range(BATCH):
                            ws.append(wbuf[ring, b, t, sl])
                    ys = []
                    for k in range(HB):
                        for b in range(BATCH):
                            x = ws[k * BATCH + b] + ps[k]
                            y = x * sb[b]
                            ys.append(y * ls[k] if with_lnw else y)
                    for k in range(HB):
                        sl = pl.ds((h0 + k) * L, L)
                        for b in range(BATCH):
                            obuf[b, t, sl] = ys[k * BATCH + b]
                return tc
            return tbody

        @pl.when(lnw_ones)
        def _():
            lax.fori_loop(0, C, make_tbody(False), 0)

        @pl.when(jnp.logical_not(lnw_ones))
        def _():
            lax.fori_loop(0, C, make_tbody(True), 0)

        for b in range(BATCH):
            pltpu.async_copy(obuf.at[b], out_dst(j, b), ssem.at[b])
        return carry

    lax.fori_loop(0, NJ, body, 0)

    for b in range(BATCH):
        pltpu.make_async_copy(obuf.at[b], out_dst(NJ - 1, b),
                              ssem.at[b]).wait()


@jax.jit
def _sc_embed(idx, words, pos, lnw):
    mesh = plsc.VectorSubcoreMesh(core_axis_name="c", subcore_axis_name="s")
    f = functools.partial(
        pl.kernel,
        mesh=mesh,
        out_type=jax.ShapeDtypeStruct((BATCH * SEQ, HIDDEN), jnp.float32),
        scratch_types=[
            pltpu.VMEM((NJ * BATCH, C), jnp.int32),
            pltpu.VMEM((2, C, HIDDEN), jnp.float32),
            pltpu.VMEM((2, BATCH, C, HIDDEN), jnp.float32),
            pltpu.VMEM((BATCH, C, HIDDEN), jnp.float32),
            pltpu.VMEM((HIDDEN,), jnp.float32),
            pltpu.SemaphoreType.DMA((2, BATCH)),
            pltpu.SemaphoreType.DMA((BATCH,)),
            pltpu.SemaphoreType.DMA((2,)),
        ],
        compiler_params=pltpu.CompilerParams(needs_layout_passes=False),
    )(_sc_embed_kernel)
    return f(idx, words, pos, lnw)


def kernel(input_ids, word_embeddings, position_embeddings, ln_weight):
    ids = input_ids.astype(jnp.int32)
    idx = ids.reshape(BATCH, NW, NJ, C).transpose(1, 2, 0, 3)
    idx = idx.reshape(NW, NJ * BATCH, C)
    out = _sc_embed(idx, word_embeddings, position_embeddings, ln_weight)
    return out.reshape(BATCH, SEQ, HIDDEN)


# speedup vs baseline: 2.9638x; 1.0097x over previous
import functools

import jax
import jax.numpy as jnp
from jax import lax
from jax.experimental import pallas as pl
from jax.experimental.pallas import tpu as pltpu
from jax.experimental.pallas import tpu_sc as plsc

VOCAB = 100000
HIDDEN = 1024
MAX_POS = 4096
BATCH = 4
SEQ = 4096
EPS = 1e-12

NW = 32
C = 8
SPW = SEQ // NW
NJ = SPW // C
L = 16
HV = HIDDEN // L
HB = 2


def _rsqrt(v):
    half = v * 0.5
    i = plsc.bitcast(v, jnp.int32)
    i = jnp.int32(0x5F3759DF) - (i >> 1)
    y = plsc.bitcast(i, jnp.float32)
    y = y * (1.5 - half * y * y)
    y = y * (1.5 - half * y * y)
    return y


def _perm(v, idx):
    return lax.gather(
        v, idx[:, None],
        dimension_numbers=lax.GatherDimensionNumbers(
            offset_dims=(), collapsed_slice_dims=(0,), start_index_map=(0,)),
        slice_sizes=(1,),
        mode=lax.GatherScatterMode.PROMISE_IN_BOUNDS)


def _lane_bcast(v, lane):
    return _perm(v, jnp.full((L,), lane, dtype=jnp.int32))


def _allsum(v, lane):
    for d in (8, 4, 2, 1):
        v = v + _perm(v, lane ^ d)
    return v


def _sc_embed_kernel(idx_hbm, words_hbm, pos_hbm, lnw_hbm, out_hbm,
                     idx_v, pbuf, wbuf, obuf, lnw_v, gsem, ssem, psem):
    wid = lax.axis_index("s") * 2 + lax.axis_index("c")
    p0 = wid * SPW

    pltpu.sync_copy(lnw_hbm, lnw_v)
    pltpu.sync_copy(idx_hbm.at[wid], idx_v)

    lane = lax.iota(jnp.int32, L)

    ones_chk = lnw_v[pl.ds(0, L)] == 1.0
    for h in range(1, HV):
        ones_chk = jnp.logical_and(ones_chk, lnw_v[pl.ds(h * L, L)] == 1.0)
    lnw_ones = jnp.all(ones_chk)

    def pos_src(j):
        return pos_hbm.at[pl.ds(p0 + j * C, C)]

    def out_dst(j, b):
        return out_hbm.at[pl.ds(b * SEQ + p0 + j * C, C)]

    def start_gathers(j, ring):
        for b in range(BATCH):
            pltpu.async_copy(words_hbm.at[idx_v.at[j * BATCH + b]],
                             wbuf.at[ring, b], gsem.at[ring, b])

    pltpu.async_copy(pos_src(0), pbuf.at[0], psem.at[0])
    start_gathers(0, 0)

    def body(j, carry):
        ring = j & 1
        pr = j & 1

        @pl.when(j + 1 < NJ)
        def _():
            start_gathers(j + 1, 1 - ring)

        pltpu.make_async_copy(pos_src(j), pbuf.at[pr], psem.at[pr]).wait()

        @pl.when(j + 1 < NJ)
        def _():
            pltpu.async_copy(pos_src(j + 1), pbuf.at[1 - pr],
                             psem.at[1 - pr])

        for b in range(BATCH):
            pltpu.make_async_copy(words_hbm.at[idx_v.at[j * BATCH + b]],
                                  wbuf.at[ring, b], gsem.at[ring, b]).wait()

        @pl.when(j >= 1)
        def _():
            for b in range(BATCH):
                pltpu.make_async_copy(obuf.at[b], out_dst(j - 1, b),
                                      ssem.at[b]).wait()

        def make_tbody(with_lnw):
            def tbody(t, tc):
                acc = [jnp.zeros((L,), jnp.float32) for _ in range(2 * BATCH)]
                for h in range(HV):
                    sl = pl.ds(h * L, L)
                    p = pbuf[pr, t, sl]
                    for b in range(BATCH):
                        x = wbuf[ring, b, t, sl] + p
                        acc[2 * b + (h & 1)] = acc[2 * b + (h & 1)] + x * x
                r = None
                for b in range(BATCH):
                    tot = _allsum(acc[2 * b] + acc[2 * b + 1], lane)
                    r = tot if b == 0 else jnp.where(lane == b, tot, r)
                s = _rsqrt(r * (1.0 / HIDDEN) + EPS)
                sb = [_lane_bcast(s, b) for b in range(BATCH)]
                def load_blk(h0):
                    ps, ls, ws = [], [], []
                    for k in range(HB):
                        sl = pl.ds((h0 + k) * L, L)
                        ps.append(pbuf[pr, t, sl])
                        if with_lnw:
                            ls.append(lnw_v[sl])
                        for b in range(BATCH):
                            ws.append(wbuf[ring, b, t, sl])
                    return ps, ls, ws

                def scale_blk(blk):
                    ps, ls, ws = blk
                    ys = []
                    for k in range(HB):
                        for b in range(BATCH):
                            x = ws[k * BATCH + b] + ps[k]
                            y = x * sb[b]
                            ys.append(y * ls[k] if with_lnw else y)
                    return ys

                def store_blk(h0, ys):
                    for k in range(HB):
                        sl = pl.ds((h0 + k) * L, L)
                        for b in range(BATCH):
                            obuf[b, t, sl] = ys[k * BATCH + b]

                blk = load_blk(0)
                for h0 in range(0, HV, HB):
                    nxt = load_blk(h0 + HB) if h0 + HB < HV else None
                    store_blk(h0, scale_blk(blk))
                    blk = nxt
                return tc
            return tbody

        @pl.when(lnw_ones)
        def _():
            lax.fori_loop(0, C, make_tbody(False), 0)

        @pl.when(jnp.logical_not(lnw_ones))
        def _():
            lax.fori_loop(0, C, make_tbody(True), 0)

        for b in range(BATCH):
            pltpu.async_copy(obuf.at[b], out_dst(j, b), ssem.at[b])
        return carry

    lax.fori_loop(0, NJ, body, 0)

    for b in range(BATCH):
        pltpu.make_async_copy(obuf.at[b], out_dst(NJ - 1, b),
                              ssem.at[b]).wait()


@jax.jit
def _sc_embed(idx, words, pos, lnw):
    mesh = plsc.VectorSubcoreMesh(core_axis_name="c", subcore_axis_name="s")
    f = functools.partial(
        pl.kernel,
        mesh=mesh,
        out_type=jax.ShapeDtypeStruct((BATCH * SEQ, HIDDEN), jnp.float32),
        scratch_types=[
            pltpu.VMEM((NJ * BATCH, C), jnp.int32),
            pltpu.VMEM((2, C, HIDDEN), jnp.float32),
            pltpu.VMEM((2, BATCH, C, HIDDEN), jnp.float32),
            pltpu.VMEM((BATCH, C, HIDDEN), jnp.float32),
            pltpu.VMEM((HIDDEN,), jnp.float32),
            pltpu.SemaphoreType.DMA((2, BATCH)),
            pltpu.SemaphoreType.DMA((BATCH,)),
            pltpu.SemaphoreType.DMA((2,)),
        ],
        compiler_params=pltpu.CompilerParams(needs_layout_passes=False),
    )(_sc_embed_kernel)
    return f(idx, words, pos, lnw)


def kernel(input_ids, word_embeddings, position_embeddings, ln_weight):
    ids = input_ids.astype(jnp.int32)
    idx = ids.reshape(BATCH, NW, NJ, C).transpose(1, 2, 0, 3)
    idx = idx.reshape(NW, NJ * BATCH, C)
    out = _sc_embed(idx, word_embeddings, position_embeddings, ln_weight)
    return out.reshape(BATCH, SEQ, HIDDEN)


# speedup vs baseline: 2.9773x; 1.0046x over previous
import functools

import jax
import jax.numpy as jnp
from jax import lax
from jax.experimental import pallas as pl
from jax.experimental.pallas import tpu as pltpu
from jax.experimental.pallas import tpu_sc as plsc

VOCAB = 100000
HIDDEN = 1024
MAX_POS = 4096
BATCH = 4
SEQ = 4096
EPS = 1e-12

NW = 32
C = 8
SPW = SEQ // NW
NJ = SPW // C
L = 16
HV = HIDDEN // L
HB = 8


def _rsqrt(v):
    half = v * 0.5
    i = plsc.bitcast(v, jnp.int32)
    i = jnp.int32(0x5F3759DF) - (i >> 1)
    y = plsc.bitcast(i, jnp.float32)
    y = y * (1.5 - half * y * y)
    y = y * (1.5 - half * y * y)
    return y


def _perm(v, idx):
    return lax.gather(
        v, idx[:, None],
        dimension_numbers=lax.GatherDimensionNumbers(
            offset_dims=(), collapsed_slice_dims=(0,), start_index_map=(0,)),
        slice_sizes=(1,),
        mode=lax.GatherScatterMode.PROMISE_IN_BOUNDS)


def _lane_bcast(v, lane):
    return _perm(v, jnp.full((L,), lane, dtype=jnp.int32))


def _allsum(v, lane):
    for d in (8, 4, 2, 1):
        v = v + _perm(v, lane ^ d)
    return v


def _sc_embed_kernel(idx_hbm, words_hbm, pos_hbm, lnw_hbm, out_hbm,
                     idx_v, pbuf, wbuf, obuf, lnw_v, gsem, ssem, psem):
    wid = lax.axis_index("s") * 2 + lax.axis_index("c")
    p0 = wid * SPW

    pltpu.sync_copy(lnw_hbm, lnw_v)
    pltpu.sync_copy(idx_hbm.at[wid], idx_v)

    lane = lax.iota(jnp.int32, L)

    ones_chk = lnw_v[pl.ds(0, L)] == 1.0
    for h in range(1, HV):
        ones_chk = jnp.logical_and(ones_chk, lnw_v[pl.ds(h * L, L)] == 1.0)
    lnw_ones = jnp.all(ones_chk)

    def pos_src(j):
        return pos_hbm.at[pl.ds(p0 + j * C, C)]

    def out_dst(j, b):
        return out_hbm.at[pl.ds(b * SEQ + p0 + j * C, C)]

    def start_gathers(j, ring):
        for b in range(BATCH):
            pltpu.async_copy(words_hbm.at[idx_v.at[j * BATCH + b]],
                             wbuf.at[ring, b], gsem.at[ring, b])

    pltpu.async_copy(pos_src(0), pbuf.at[0], psem.at[0])
    start_gathers(0, 0)

    def body(j, carry):
        ring = j & 1
        pr = j & 1

        @pl.when(j + 1 < NJ)
        def _():
            start_gathers(j + 1, 1 - ring)

        pltpu.make_async_copy(pos_src(j), pbuf.at[pr], psem.at[pr]).wait()

        @pl.when(j + 1 < NJ)
        def _():
            pltpu.async_copy(pos_src(j + 1), pbuf.at[1 - pr],
                             psem.at[1 - pr])

        for b in range(BATCH):
            pltpu.make_async_copy(words_hbm.at[idx_v.at[j * BATCH + b]],
                                  wbuf.at[ring, b], gsem.at[ring, b]).wait()

        @pl.when(j >= 1)
        def _():
            for b in range(BATCH):
                pltpu.make_async_copy(obuf.at[b], out_dst(j - 1, b),
                                      ssem.at[b]).wait()

        def make_tbody(with_lnw):
            def tbody(t, tc):
                acc = [jnp.zeros((L,), jnp.float32) for _ in range(2 * BATCH)]
                for h in range(HV):
                    sl = pl.ds(h * L, L)
                    p = pbuf[pr, t, sl]
                    for b in range(BATCH):
                        x = wbuf[ring, b, t, sl] + p
                        acc[2 * b + (h & 1)] = acc[2 * b + (h & 1)] + x * x
                r = None
                for b in range(BATCH):
                    tot = _allsum(acc[2 * b] + acc[2 * b + 1], lane)
                    r = tot if b == 0 else jnp.where(lane == b, tot, r)
                s = _rsqrt(r * (1.0 / HIDDEN) + EPS)
                sb = [_lane_bcast(s, b) for b in range(BATCH)]
                def load_blk(h0):
                    ps, ls, ws = [], [], []
                    for k in range(HB):
                        sl = pl.ds((h0 + k) * L, L)
                        ps.append(pbuf[pr, t, sl])
                        if with_lnw:
                            ls.append(lnw_v[sl])
                        for b in range(BATCH):
                            ws.append(wbuf[ring, b, t, sl])
                    return ps, ls, ws

                def scale_blk(blk):
                    ps, ls, ws = blk
                    ys = []
                    for k in range(HB):
                        for b in range(BATCH):
                            x = ws[k * BATCH + b] + ps[k]
                            y = x * sb[b]
                            ys.append(y * ls[k] if with_lnw else y)
                    return ys

                def store_blk(h0, ys):
                    for k in range(HB):
                        sl = pl.ds((h0 + k) * L, L)
                        for b in range(BATCH):
                            obuf[b, t, sl] = ys[k * BATCH + b]

                blk = load_blk(0)
                for h0 in range(0, HV, HB):
                    nxt = load_blk(h0 + HB) if h0 + HB < HV else None
                    store_blk(h0, scale_blk(blk))
                    blk = nxt
                return tc
            return tbody

        @pl.when(lnw_ones)
        def _():
            lax.fori_loop(0, C, make_tbody(False), 0)

        @pl.when(jnp.logical_not(lnw_ones))
        def _():
            lax.fori_loop(0, C, make_tbody(True), 0)

        for b in range(BATCH):
            pltpu.async_copy(obuf.at[b], out_dst(j, b), ssem.at[b])
        return carry

    lax.fori_loop(0, NJ, body, 0)

    for b in range(BATCH):
        pltpu.make_async_copy(obuf.at[b], out_dst(NJ - 1, b),
                              ssem.at[b]).wait()


@jax.jit
def _sc_embed(idx, words, pos, lnw):
    mesh = plsc.VectorSubcoreMesh(core_axis_name="c", subcore_axis_name="s")
    f = functools.partial(
        pl.kernel,
        mesh=mesh,
        out_type=jax.ShapeDtypeStruct((BATCH * SEQ, HIDDEN), jnp.float32),
        scratch_types=[
            pltpu.VMEM((NJ * BATCH, C), jnp.int32),
            pltpu.VMEM((2, C, HIDDEN), jnp.float32),
            pltpu.VMEM((2, BATCH, C, HIDDEN), jnp.float32),
            pltpu.VMEM((BATCH, C, HIDDEN), jnp.float32),
            pltpu.VMEM((HIDDEN,), jnp.float32),
            pltpu.SemaphoreType.DMA((2, BATCH)),
            pltpu.SemaphoreType.DMA((BATCH,)),
            pltpu.SemaphoreType.DMA((2,)),
        ],
        compiler_params=pltpu.CompilerParams(needs_layout_passes=False),
    )(_sc_embed_kernel)
    return f(idx, words, pos, lnw)


def kernel(input_ids, word_embeddings, position_embeddings, ln_weight):
    ids = input_ids.astype(jnp.int32)
    idx = ids.reshape(BATCH, NW, NJ, C).transpose(1, 2, 0, 3)
    idx = idx.reshape(NW, NJ * BATCH, C)
    out = _sc_embed(idx, word_embeddings, position_embeddings, ln_weight)
    return out.reshape(BATCH, SEQ, HIDDEN)
